# baseline (speedup 1.0000x reference)
import functools

import jax
import jax.numpy as jnp
from jax import lax
from jax.experimental import pallas as pl
from jax.experimental.pallas import tpu as pltpu
from jax.experimental.pallas import tpu_sc as plsc

VOCAB = 1000
DIM = 300
BATCH = 4096
HIST = 50
NUM_FLAT = BATCH * HIST

LANES = 16
NUM_WORKERS = 32
CHUNK = NUM_FLAT // NUM_WORKERS
HALF = CHUNK // 2
EX_PER_WORKER = BATCH // NUM_WORKERS

SLAB = 128
NUM_SLABS = 8
G_ROWS = NUM_SLABS * VOCAB


def _tc_body(ct_ref, xt_ref, g_ref):
    ct = lax.transpose(ct_ref[...], (1, 0))
    xt = jnp.pad(xt_ref[...], ((0, 0), (0, NUM_SLABS * SLAB - VOCAB)))
    g = lax.dot_general(
        ct,
        xt,
        (((1,), (0,)), ((), ())),
        preferred_element_type=jnp.float32,
        precision=lax.Precision.DEFAULT,
    )
    for c in range(NUM_SLABS):
        g_ref[pl.ds(c * VOCAB, VOCAB), :] = g[:, c * SLAB : (c + 1) * SLAB]


def _tc_stage(center_table_t, context_table_t):
    return pl.pallas_call(
        _tc_body,
        out_shape=jax.ShapeDtypeStruct((G_ROWS, SLAB), jnp.float32),
    )(center_table_t, context_table_t)


def _sc_gather(g16, cid, ctx_flat):
    mesh = plsc.VectorSubcoreMesh(core_axis_name="core", subcore_axis_name="subcore")

    @functools.partial(
        pl.kernel,
        out_type=jax.ShapeDtypeStruct((NUM_FLAT,), jnp.float32),
        mesh=mesh,
        scratch_types=[
            pltpu.VMEM((EX_PER_WORKER,), jnp.int32),
            pltpu.VMEM((CHUNK,), jnp.int32),
            pltpu.VMEM((HALF,), jnp.int32),
            pltpu.VMEM((HALF,), jnp.int32),
            pltpu.VMEM((HALF, LANES), jnp.float32),
            pltpu.VMEM((HALF, LANES), jnp.float32),
            pltpu.VMEM((CHUNK,), jnp.float32),
            pltpu.SemaphoreType.DMA,
            pltpu.SemaphoreType.DMA,
        ],
        compiler_params=pltpu.CompilerParams(
            use_tc_tiling_on_sc=False,
            needs_layout_passes=False,
            skip_device_barrier=True,
        ),
    )
    def k(g_hbm, cid_hbm, ctx_hbm, o_hbm,
          cid_v, ctx_v, row_a, row_b, rows_a, rows_b, out_v, sem_a, sem_b):
        wid = lax.axis_index("subcore") * 2 + lax.axis_index("core")
        base = wid * CHUNK
        pltpu.sync_copy(cid_hbm.at[pl.ds(wid * EX_PER_WORKER, EX_PER_WORKER)], cid_v)
        pltpu.sync_copy(ctx_hbm.at[pl.ds(base, CHUNK)], ctx_v)

        def compute_rows(off, row_ref):
            @plsc.parallel_loop(0, HALF, step=LANES, unroll=2)
            def _(c):
                p = lax.iota(jnp.int32, LANES) + (off + c)
                cidv = plsc.load_gather(cid_v, [p // HIST])
                ctxv = ctx_v[pl.ds(off + c, LANES)]
                row_ref[pl.ds(c, LANES)] = (
                    (lax.shift_right_logical(ctxv, 7) * VOCAB + cidv) * 8
                    + lax.bitwise_and(lax.shift_right_logical(ctxv, 4), 7)
                )

        def select(off, rows_ref):
            @plsc.parallel_loop(0, HALF, step=LANES, unroll=2)
            def _(c):
                lane = lax.bitwise_and(ctx_v[pl.ds(off + c, LANES)], LANES - 1)
                jvec = lax.iota(jnp.int32, LANES) + c
                out_v[pl.ds(off + c, LANES)] = plsc.load_gather(
                    rows_ref, [jvec, lane]
                )

        compute_rows(0, row_a)
        dma_a = pltpu.async_copy(g_hbm.at[row_a], rows_a, sem_a)
        compute_rows(HALF, row_b)
        dma_b = pltpu.async_copy(g_hbm.at[row_b], rows_b, sem_b)
        dma_a.wait()
        select(0, rows_a)
        dma_b.wait()
        select(HALF, rows_b)

        pltpu.sync_copy(out_v, o_hbm.at[pl.ds(base, CHUNK)])

    return k(g16, cid, ctx_flat)


def kernel(center_ids, context_ids, center_table, context_table):
    g_hat = _tc_stage(center_table.T, context_table.T)
    scores = _sc_gather(
        g_hat.reshape(G_ROWS * SLAB // LANES, LANES),
        center_ids,
        context_ids.reshape(NUM_FLAT),
    )
    return scores.reshape(BATCH, HIST)


# speedup vs baseline: 1.0269x; 1.0269x over previous
import functools

import jax
import jax.numpy as jnp
from jax import lax
from jax.experimental import pallas as pl
from jax.experimental.pallas import tpu as pltpu
from jax.experimental.pallas import tpu_sc as plsc

VOCAB = 1000
DIM = 300
BATCH = 4096
HIST = 50
NUM_FLAT = BATCH * HIST

LANES = 16
NUM_WORKERS = 32
CHUNK = NUM_FLAT // NUM_WORKERS
NSTAGE = 4
QUARTER = CHUNK // NSTAGE
EX_PER_WORKER = BATCH // NUM_WORKERS

SLAB = 128
NUM_SLABS = 8
G_ROWS = NUM_SLABS * VOCAB


def _tc_body(ct_ref, xt_ref, g_ref):
    ct = lax.transpose(ct_ref[...], (1, 0))
    xt = jnp.pad(xt_ref[...], ((0, 0), (0, NUM_SLABS * SLAB - VOCAB)))
    g = lax.dot_general(
        ct,
        xt,
        (((1,), (0,)), ((), ())),
        preferred_element_type=jnp.float32,
        precision=lax.Precision.DEFAULT,
    )
    for c in range(NUM_SLABS):
        g_ref[pl.ds(c * VOCAB, VOCAB), :] = g[:, c * SLAB : (c + 1) * SLAB]


def _tc_stage(center_table_t, context_table_t):
    return pl.pallas_call(
        _tc_body,
        out_shape=jax.ShapeDtypeStruct((G_ROWS, SLAB), jnp.float32),
    )(center_table_t, context_table_t)


def _sc_gather(g16, cid, ctx_flat):
    mesh = plsc.VectorSubcoreMesh(core_axis_name="core", subcore_axis_name="subcore")

    @functools.partial(
        pl.kernel,
        out_type=jax.ShapeDtypeStruct((HIST, BATCH), jnp.float32),
        mesh=mesh,
        scratch_types=[
            pltpu.VMEM((EX_PER_WORKER,), jnp.int32),
            pltpu.VMEM((CHUNK,), jnp.int32),
            [pltpu.VMEM((QUARTER,), jnp.int32) for _ in range(NSTAGE)],
            [pltpu.VMEM((QUARTER, LANES), jnp.float32) for _ in range(NSTAGE)],
            pltpu.VMEM((HIST, EX_PER_WORKER), jnp.float32),
            [pltpu.SemaphoreType.DMA for _ in range(NSTAGE)],
        ],
        compiler_params=pltpu.CompilerParams(
            use_tc_tiling_on_sc=False,
            needs_layout_passes=False,
            skip_device_barrier=True,
        ),
    )
    def k(g_hbm, cid_hbm, ctx_hbm, o_hbm, cid_v, ctx_v, row_q, rows_q, out_v, sems):
        wid = lax.axis_index("subcore") * 2 + lax.axis_index("core")
        base = wid * CHUNK
        pltpu.sync_copy(cid_hbm.at[pl.ds(wid * EX_PER_WORKER, EX_PER_WORKER)], cid_v)
        pltpu.sync_copy(ctx_hbm.at[pl.ds(base, CHUNK)], ctx_v)

        def compute_rows(off, row_ref):
            @plsc.parallel_loop(0, QUARTER, step=LANES, unroll=8)
            def _(c):
                p = lax.iota(jnp.int32, LANES) + (off + c)
                cidv = plsc.load_gather(cid_v, [p // HIST])
                ctxv = ctx_v[pl.ds(off + c, LANES)]
                row_ref[pl.ds(c, LANES)] = (
                    (lax.shift_right_logical(ctxv, 7) * VOCAB + cidv) * 8
                    + lax.bitwise_and(lax.shift_right_logical(ctxv, 4), 7)
                )

        def select(off, rows_ref):
            @plsc.parallel_loop(0, QUARTER, step=LANES, unroll=8)
            def _(c):
                p = lax.iota(jnp.int32, LANES) + (off + c)
                lane = lax.bitwise_and(ctx_v[pl.ds(off + c, LANES)], LANES - 1)
                jvec = lax.iota(jnp.int32, LANES) + c
                vals = plsc.load_gather(rows_ref, [jvec, lane])
                plsc.store_scatter(out_v, [p % HIST, p // HIST], vals)

        dmas = []
        for s in range(NSTAGE):
            compute_rows(s * QUARTER, row_q[s])
            dmas.append(pltpu.async_copy(g_hbm.at[row_q[s]], rows_q[s], sems[s]))
        for s in range(NSTAGE):
            dmas[s].wait()
            select(s * QUARTER, rows_q[s])

        pltpu.sync_copy(out_v, o_hbm.at[:, pl.ds(wid * EX_PER_WORKER, EX_PER_WORKER)])

    return k(g16, cid, ctx_flat)


def kernel(center_ids, context_ids, center_table, context_table):
    g_hat = _tc_stage(center_table.T, context_table.T)
    scores_t = _sc_gather(
        g_hat.reshape(G_ROWS * SLAB // LANES, LANES),
        center_ids,
        context_ids.reshape(NUM_FLAT),
    )
    return scores_t.T


# speedup vs baseline: 1.0811x; 1.0527x over previous
import functools

import jax
import jax.numpy as jnp
from jax import lax
from jax.experimental import pallas as pl
from jax.experimental.pallas import tpu as pltpu
from jax.experimental.pallas import tpu_sc as plsc

VOCAB = 1000
DIM = 300
BATCH = 4096
HIST = 50
NUM_FLAT = BATCH * HIST

LANES = 16
NUM_WORKERS = 32
CHUNK = NUM_FLAT // NUM_WORKERS
NSTAGE = 2
QUARTER = CHUNK // NSTAGE
EX_PER_WORKER = BATCH // NUM_WORKERS

SLAB = 128
NUM_SLABS = 8
G_ROWS = NUM_SLABS * VOCAB


def _tc_body(ct_ref, xt_ref, g_ref):
    ct = lax.transpose(ct_ref[...], (1, 0))
    xt = jnp.pad(xt_ref[...], ((0, 0), (0, NUM_SLABS * SLAB - VOCAB)))
    g = lax.dot_general(
        ct,
        xt,
        (((1,), (0,)), ((), ())),
        preferred_element_type=jnp.float32,
        precision=lax.Precision.DEFAULT,
    )
    for c in range(NUM_SLABS):
        g_ref[pl.ds(c * VOCAB, VOCAB), :] = g[:, c * SLAB : (c + 1) * SLAB]


def _tc_stage(center_table_t, context_table_t):
    return pl.pallas_call(
        _tc_body,
        out_shape=jax.ShapeDtypeStruct((G_ROWS, SLAB), jnp.float32),
    )(center_table_t, context_table_t)


def _sc_gather(g16, cid, ctx_flat):
    mesh = plsc.VectorSubcoreMesh(core_axis_name="core", subcore_axis_name="subcore")

    @functools.partial(
        pl.kernel,
        out_type=jax.ShapeDtypeStruct((HIST, BATCH), jnp.float32),
        mesh=mesh,
        scratch_types=[
            pltpu.VMEM((EX_PER_WORKER,), jnp.int32),
            pltpu.VMEM((CHUNK,), jnp.int32),
            [pltpu.VMEM((QUARTER,), jnp.int32) for _ in range(NSTAGE)],
            [pltpu.VMEM((QUARTER, LANES), jnp.float32) for _ in range(NSTAGE)],
            pltpu.VMEM((HIST, EX_PER_WORKER), jnp.float32),
            [pltpu.SemaphoreType.DMA for _ in range(NSTAGE)],
        ],
        compiler_params=pltpu.CompilerParams(
            use_tc_tiling_on_sc=False,
            needs_layout_passes=False,
            skip_device_barrier=True,
        ),
    )
    def k(g_hbm, cid_hbm, ctx_hbm, o_hbm, cid_v, ctx_v, row_q, rows_q, out_v, sems):
        wid = lax.axis_index("subcore") * 2 + lax.axis_index("core")
        base = wid * CHUNK
        pltpu.sync_copy(cid_hbm.at[pl.ds(wid * EX_PER_WORKER, EX_PER_WORKER)], cid_v)
        pltpu.sync_copy(ctx_hbm.at[pl.ds(base, CHUNK)], ctx_v)

        iota = lax.iota(jnp.int32, LANES)
        iota_h = iota * HIST
        b16s = EX_PER_WORKER // LANES // NSTAGE

        def compute_rows(stage, row_ref):
            @plsc.parallel_loop(0, HIST, step=1, unroll=2)
            def _(l):
                for g in range(b16s):
                    boff = (stage * b16s + g) * LANES
                    ctxv = plsc.load_gather(ctx_v, [iota_h + (boff * HIST + l)])
                    cidv = cid_v[pl.ds(boff, LANES)]
                    row_ref[pl.ds(l * (b16s * LANES) + g * LANES, LANES)] = (
                        (lax.shift_right_logical(ctxv, 7) * VOCAB + cidv) * 8
                        + lax.bitwise_and(lax.shift_right_logical(ctxv, 4), 7)
                    )

        def select(stage, rows_ref):
            @plsc.parallel_loop(0, HIST, step=1, unroll=2)
            def _(l):
                for g in range(b16s):
                    boff = (stage * b16s + g) * LANES
                    ctxv = plsc.load_gather(ctx_v, [iota_h + (boff * HIST + l)])
                    lane = lax.bitwise_and(ctxv, LANES - 1)
                    jvec = iota + (l * (b16s * LANES) + g * LANES)
                    vals = plsc.load_gather(rows_ref, [jvec, lane])
                    out_v[l, pl.ds(boff, LANES)] = vals

        dmas = []
        for s in range(NSTAGE):
            compute_rows(s, row_q[s])
            dmas.append(pltpu.async_copy(g_hbm.at[row_q[s]], rows_q[s], sems[s]))
        for s in range(NSTAGE):
            dmas[s].wait()
            select(s, rows_q[s])

        pltpu.sync_copy(out_v, o_hbm.at[:, pl.ds(wid * EX_PER_WORKER, EX_PER_WORKER)])

    return k(g16, cid, ctx_flat)


def kernel(center_ids, context_ids, center_table, context_table):
    g_hat = _tc_stage(center_table.T, context_table.T)
    scores_t = _sc_gather(
        g_hat.reshape(G_ROWS * SLAB // LANES, LANES),
        center_ids,
        context_ids.reshape(NUM_FLAT),
    )
    return scores_t.T
